# baseline (speedup 1.0000x reference)
import functools

import jax
import jax.numpy as jnp
from jax import lax
from jax.experimental import pallas as pl
from jax.experimental.pallas import tpu as pltpu
from jax.experimental.pallas import tpu_sc as plsc

H = 768
S = 77
B = 4096
NC, NS = 2, 16
SEQ_PER_SUB = B // NS
LANES = 16
NROW = 40
ROW_SPLIT = ((0, NROW), (NROW, S - NROW))
CH = 32
NBUF = 3
LOOP_SLOTS = SEQ_PER_SUB - 1

_mesh = plsc.VectorSubcoreMesh(core_axis_name="c", subcore_axis_name="s")


@functools.partial(
    pl.kernel,
    out_type=jax.ShapeDtypeStruct((B, S, H), jnp.float32),
    mesh=_mesh,
    compiler_params=pltpu.CompilerParams(use_tc_tiling_on_sc=False),
    scratch_types=[
        pltpu.VMEM((2, CH, 1, S), jnp.int32),
        pltpu.VMEM((NROW, H), jnp.float32),
        pltpu.VMEM((NROW, H), jnp.float32),
        pltpu.VMEM((NROW, H), jnp.float32),
        pltpu.VMEM((NROW, H), jnp.float32),
        pltpu.SemaphoreType.DMA,
        pltpu.SemaphoreType.DMA,
        pltpu.SemaphoreType.DMA,
        pltpu.SemaphoreType.DMA,
        pltpu.SemaphoreType.DMA,
        pltpu.SemaphoreType.DMA,
    ],
)
def _embed(ids_hbm, tab_hbm, pos_hbm, out_hbm,
           idx_v, pos_v, buf0, buf1, buf2, g0, g1, g2, so0, so1, so2):
    c = lax.axis_index("c")
    sid = lax.axis_index("s")
    seq0 = sid * SEQ_PER_SUB

    bufs = (buf0, buf1, buf2)
    gsem = (g0, g1, g2)
    ssem = (so0, so1, so2)

    def refresh(j):
        pltpu.sync_copy(ids_hbm.at[pl.ds(seq0 + j, CH)],
                        idx_v.at[(j // CH) % 2])

    for ci in range(NC):
        r0, nr = ROW_SPLIT[ci]

        @pl.when(c == ci)
        def _():
            pltpu.sync_copy(pos_hbm.at[pl.ds(r0, nr)], pos_v.at[pl.ds(0, nr)])

            def gstart(j, b):
                pltpu.async_copy(
                    tab_hbm.at[idx_v.at[(j // CH) % 2, j % CH, 0,
                                        pl.ds(r0, nr)]],
                    bufs[b].at[pl.ds(0, nr)], gsem[b])

            def gwait(b):
                pltpu.make_async_copy(
                    tab_hbm.at[pl.ds(0, nr)],
                    bufs[b].at[pl.ds(0, nr)], gsem[b]).wait()

            def sstart(j, b):
                pltpu.async_copy(
                    bufs[b].at[pl.ds(0, nr)],
                    out_hbm.at[seq0 + j, pl.ds(r0, nr)], ssem[b])

            def swait(j, b):
                pltpu.make_async_copy(
                    bufs[b].at[pl.ds(0, nr)],
                    out_hbm.at[seq0 + j, pl.ds(r0, nr)], ssem[b]).wait()

            def add_pos(b):
                def add_row(r, c2):
                    for g in range(H // LANES):
                        sl = pl.ds(g * LANES, LANES)
                        plsc.addupdate(bufs[b].at[r, sl], pos_v[r, sl])
                    return c2
                lax.fori_loop(0, nr, add_row, 0)

            refresh(0)
            gstart(0, 0)
            gstart(1, 1)

            def outer(i2, carry):
                for b in range(NBUF):
                    i = i2 * NBUF + b
                    nb = (b + 2) % NBUF
                    gwait(b)

                    @pl.when(i >= 1)
                    def _():
                        swait(i - 1, nb)

                    @pl.when(jnp.logical_and((i + 2) % CH == 0,
                                             i + 2 < LOOP_SLOTS))
                    def _():
                        refresh(i + 2)

                    @pl.when(i + 2 <= LOOP_SLOTS - 1)
                    def _():
                        gstart(i + 2, nb)

                    add_pos(b)
                    sstart(i, b)
                return carry

            lax.fori_loop(0, LOOP_SLOTS // NBUF, outer, 0)

            gstart(SEQ_PER_SUB - 1, 0)
            gwait(0)
            add_pos(0)
            sstart(SEQ_PER_SUB - 1, 0)
            swait(SEQ_PER_SUB - 2, 2)
            swait(SEQ_PER_SUB - 1, 0)


def kernel(input_ids, token_embedding, position_embedding):
    ids3 = input_ids.reshape(B, 1, S)
    return _embed(ids3, token_embedding, position_embedding)
